# baseline (speedup 1.0000x reference)
import functools

import jax
import jax.numpy as jnp
from jax import lax
from jax.experimental import pallas as pl
from jax.experimental.pallas import tpu as pltpu
from jax.experimental.pallas import tpu_sc as plsc

_B = 4096
_ED = 65
_DIM = 64
_BB = 1024
_NBLK = _B // _BB
_NROWS = 100000
_NPAD = 100352


def _sc_dims():
    try:
        info = plsc.get_sparse_core_info()
        return info.num_cores, info.num_subcores
    except Exception:
        return 2, 16


def _make_sc_gather():
    nc, ns = _sc_dims()
    nw = nc * ns
    bpw = _B // nw
    epw = bpw * _ED
    nchunk = epw // 16

    mesh = plsc.VectorSubcoreMesh(core_axis_name="c", subcore_axis_name="s")

    @functools.partial(
        pl.kernel,
        mesh=mesh,
        out_type=jax.ShapeDtypeStruct((_ED * _B,), jnp.float32),
        scratch_types=[
            pltpu.VMEM((bpw,), jnp.int32),
            pltpu.VMEM((epw,), jnp.int32),
            pltpu.VMEM((epw,), jnp.float32),
            pltpu.SemaphoreType.DMA,
            pltpu.SemaphoreType.DMA,
        ],
        compiler_params=pltpu.CompilerParams(needs_layout_passes=False),
    )
    def sc_gather(idx_hbm, tab_hbm, out_hbm, idx_v, eidx_v, rows_v, gsem, osem):
        wid = lax.axis_index("s") * nc + lax.axis_index("c")
        base = wid * bpw
        pltpu.sync_copy(idx_hbm.at[pl.ds(base, bpw)], idx_v)

        iota = lax.iota(jnp.int32, 16)

        def build(i, _):
            q = lax.add(lax.broadcast(lax.mul(i, 16), (16,)), iota)
            c = lax.div(q, lax.broadcast(bpw, (16,)))
            j = lax.sub(q, lax.mul(c, lax.broadcast(bpw, (16,))))
            rv = plsc.load_gather(idx_v, [j])
            eidx_v[pl.ds(i * 16, 16)] = lax.add(
                lax.mul(c, lax.broadcast(_NPAD, (16,))), rv)
            return 0

        lax.fori_loop(0, nchunk, build, 0)
        gathers = [
            pltpu.async_copy(
                tab_hbm.at[eidx_v.at[pl.ds(c * bpw, bpw)]],
                rows_v.at[pl.ds(c * bpw, bpw)], gsem)
            for c in range(_ED)
        ]
        for g in gathers:
            g.wait()
        outs = [
            pltpu.async_copy(
                rows_v.at[pl.ds(c * bpw, bpw)],
                out_hbm.at[pl.ds(c * _B + base, bpw)], osem)
            for c in range(_ED)
        ]
        for o in outs:
            o.wait()

    return sc_gather


def _tc_body(uwT_ref, iwT_ref, ageT_ref, genT_ref, occT_ref, kidT_ref, labT_ref,
             atT_ref, gtT_ref, otT_ref, ktT_ref,
             w1T_ref, b1_ref, w2T_ref, b2_ref, w3T_ref, b3_ref, w4T_ref, b4_ref,
             loss_ref, pT_ref):
    i = pl.program_id(0)
    f32 = jnp.float32

    uT = uwT_ref[...]
    iT = iwT_ref[...]

    aohT = (ageT_ref[...] == lax.broadcasted_iota(jnp.int32, (8, _BB), 0)).astype(f32)
    gohT = (genT_ref[...] == lax.broadcasted_iota(jnp.int32, (3, _BB), 0)).astype(f32)
    oohT = (occT_ref[...] == lax.broadcasted_iota(jnp.int32, (25, _BB), 0)).astype(f32)
    awT = jnp.dot(atT_ref[...], aohT, preferred_element_type=f32)
    gwT = jnp.dot(gtT_ref[...], gohT, preferred_element_type=f32)
    owT = jnp.dot(otT_ref[...], oohT, preferred_element_type=f32)

    kidT = kidT_ref[...]
    kiota = lax.broadcasted_iota(jnp.int32, (20, _BB), 0)
    ktT = ktT_ref[...]
    kwTs = []
    for j in range(10):
        kj = kidT[j:j + 1, :]
        kohT = ((kj == kiota) & (kj != 0)).astype(f32)
        kwTs.append(jnp.dot(ktT, kohT, preferred_element_type=f32))

    fieldsT = [uT, iT, awT, owT] + kwTs + [gwT]
    oneT = fieldsT[0][0:1, :]
    for fld in fieldsT[1:]:
        oneT = oneT + fld[0:1, :]

    esT = [fld[1:, :] for fld in fieldsT]
    sT = esT[0]
    for e in esT[1:]:
        sT = sT + e
    sumsqT = jnp.sum(esT[0] * esT[0], axis=0, keepdims=True)
    for e in esT[1:]:
        sumsqT = sumsqT + jnp.sum(e * e, axis=0, keepdims=True)
    twoT = 0.5 * (jnp.sum(sT * sT, axis=0, keepdims=True) - sumsqT)

    h0T = jnp.concatenate(esT, axis=0)
    hT = jnp.maximum(jnp.dot(w1T_ref[...], h0T, preferred_element_type=f32) + b1_ref[...], 0.0)
    hT = jnp.maximum(jnp.dot(w2T_ref[...], hT, preferred_element_type=f32) + b2_ref[...], 0.0)
    hT = jnp.maximum(jnp.dot(w3T_ref[...], hT, preferred_element_type=f32) + b3_ref[...], 0.0)
    mT = jnp.dot(w4T_ref[...], hT, preferred_element_type=f32) + b4_ref[...]

    logitT = oneT + twoT + mT
    pT = 1.0 / (1.0 + jnp.exp(-logitT))
    pT_ref[...] = pT

    labT = labT_ref[...]
    ploss = jnp.sum(-(labT * jnp.log(pT + 1e-6)
                      + (1.0 - labT) * jnp.log(1.0 - pT + 1e-6)),
                    axis=(0, 1), keepdims=True)

    @pl.when(i == 0)
    def _init():
        loss_ref[...] = jnp.zeros((1, 1), jnp.float32)

    loss_ref[...] += ploss

    @pl.when(i == _NBLK - 1)
    def _final():
        loss_ref[...] = loss_ref[...] * (1.0 / _B)


def _tc_main(uwT, iwT, ageT, genT, occT, kidT, labT,
             atT, gtT, otT, ktT,
             W1T, b1, W2T, b2, W3T, b3, W4T, b4):
    bspec = lambda shp: pl.BlockSpec(shp, lambda i: (0, i))
    fspec = lambda shp: pl.BlockSpec(shp, lambda i: (0, 0))
    grid_spec = pl.GridSpec(
        grid=(_NBLK,),
        in_specs=[
            bspec((_ED, _BB)), bspec((_ED, _BB)),
            bspec((1, _BB)), bspec((1, _BB)), bspec((1, _BB)), bspec((10, _BB)),
            bspec((1, _BB)),
            fspec((_ED, 8)), fspec((_ED, 3)), fspec((_ED, 25)), fspec((_ED, 20)),
            fspec((128, 960)), fspec((128, 1)),
            fspec((64, 128)), fspec((64, 1)),
            fspec((32, 64)), fspec((32, 1)),
            fspec((1, 32)), fspec((1, 1)),
        ],
        out_specs=[
            fspec((1, 1)),
            bspec((1, _BB)),
        ],
    )
    loss, pT = pl.pallas_call(
        _tc_body,
        grid_spec=grid_spec,
        out_shape=[
            jax.ShapeDtypeStruct((1, 1), jnp.float32),
            jax.ShapeDtypeStruct((1, _B), jnp.float32),
        ],
    )(uwT, iwT, ageT, genT, occT, kidT, labT,
      atT, gtT, otT, ktT,
      W1T, b1, W2T, b2, W3T, b3, W4T, b4)
    return loss, pT


def kernel(userid, itemid, user_age, gender, user_occupation, item_kind, label,
           user_table, item_table, age_table, gender_table, occ_table, kind_table,
           W1, b1, W2, b2, W3, b3, W4, b4):
    uidx = userid.reshape(_B).astype(jnp.int32)
    iidx = itemid.reshape(_B).astype(jnp.int32)
    sc_gather = _make_sc_gather()
    pad = ((0, 0), (0, _NPAD - _NROWS))
    uwT = sc_gather(uidx, jnp.pad(user_table.T, pad).reshape(-1)).reshape(_ED, _B)
    iwT = sc_gather(iidx, jnp.pad(item_table.T, pad).reshape(-1)).reshape(_ED, _B)

    loss, pT = _tc_main(
        uwT, iwT,
        user_age.astype(jnp.int32).reshape(1, _B),
        gender.astype(jnp.int32).reshape(1, _B),
        user_occupation.astype(jnp.int32).reshape(1, _B),
        item_kind.astype(jnp.int32).T,
        label.reshape(1, _B),
        age_table.T, gender_table.T, occ_table.T, kind_table.T,
        W1.T, b1.reshape(128, 1), W2.T, b2.reshape(64, 1),
        W3.T, b3.reshape(32, 1), W4.T, b4.reshape(1, 1),
    )
    return loss.reshape(()), pT.reshape(_B, 1)
